# baseline (speedup 1.0000x reference)
import jax
import jax.numpy as jnp
from jax.experimental import pallas as pl
from jax.experimental.pallas import tpu as pltpu

NUM_EXPERTS = 8
HIDDEN = 768
BLOCK_T = 2048
NSPLIT = 2


def _router_body(x0_ref, x1_ref, w_ref, p_ref, oh_ref):
    for j, x_ref in enumerate((x0_ref, x1_ref)):
        logits = jnp.dot(x_ref[...], w_ref[...], preferred_element_type=jnp.float32)
        m = jnp.max(logits, axis=-1, keepdims=True)
        unn = jnp.exp(logits - m)
        s = jnp.sum(unn, axis=-1, keepdims=True)
        probs = unn / s
        sl = pl.ds(j * BLOCK_T, BLOCK_T)
        p_ref[sl, :] = jnp.max(probs, axis=-1, keepdims=True)
        idx = jnp.argmax(probs, axis=-1)
        iota = jax.lax.broadcasted_iota(jnp.int32, probs.shape, 1)
        oh_ref[sl, :] = (iota == idx[:, None]).astype(jnp.int32)


def kernel(hidden_states, W):
    B, S, H = hidden_states.shape
    T = B * S
    x = hidden_states.reshape(T, H)
    grid = (T // (BLOCK_T * NSPLIT),)
    probs, onehot = pl.pallas_call(
        _router_body,
        grid=grid,
        in_specs=[
            pl.BlockSpec((BLOCK_T, H), lambda i: (NSPLIT * i, 0)),
            pl.BlockSpec((BLOCK_T, H), lambda i: (NSPLIT * i + 1, 0)),
            pl.BlockSpec((H, NUM_EXPERTS), lambda i: (0, 0)),
        ],
        out_specs=[
            pl.BlockSpec((BLOCK_T * NSPLIT, 1), lambda i: (i, 0)),
            pl.BlockSpec((BLOCK_T * NSPLIT, NUM_EXPERTS), lambda i: (i, 0)),
        ],
        out_shape=[
            jax.ShapeDtypeStruct((T, 1), jnp.float32),
            jax.ShapeDtypeStruct((T, NUM_EXPERTS), jnp.int32),
        ],
        compiler_params=pltpu.CompilerParams(
            dimension_semantics=("parallel",),
        ),
    )(x, x, W)
    p_out = probs.reshape(B, S, 1)
    oh_out = onehot.reshape(B, S, 1, NUM_EXPERTS).astype(jnp.int64)
    return (p_out, oh_out, p_out)


# speedup vs baseline: 1.0919x; 1.0919x over previous
import jax
import jax.numpy as jnp
from jax.experimental import pallas as pl
from jax.experimental.pallas import tpu as pltpu

NUM_EXPERTS = 8
HIDDEN = 768
CHUNK = 1024
NBUF = 4


def _router_body(x_hbm, w_ref, p_ref, oh_ref, buf, sems):
    i = pl.program_id(0)
    nc = pl.num_programs(0)

    def start(c):
        slot = jax.lax.rem(c, NBUF)
        pltpu.make_async_copy(
            x_hbm.at[pl.ds(c * CHUNK, CHUNK), :],
            buf.at[slot],
            sems.at[slot],
        ).start()

    @pl.when(i == 0)
    def _prime():
        for k in range(NBUF):
            start(k)

    slot = jax.lax.rem(i, NBUF)
    pltpu.make_async_copy(
        x_hbm.at[pl.ds(i * CHUNK, CHUNK), :], buf.at[slot], sems.at[slot]
    ).wait()

    logits = jnp.dot(buf[slot], w_ref[...], preferred_element_type=jnp.float32)
    m = jnp.max(logits, axis=-1, keepdims=True)
    unn = jnp.exp(logits - m)
    s = jnp.sum(unn, axis=-1, keepdims=True)
    probs = unn / s
    p_ref[...] = jnp.max(probs, axis=-1, keepdims=True)
    idx = jnp.argmax(probs, axis=-1)
    iota = jax.lax.broadcasted_iota(jnp.int32, probs.shape, 1)
    oh_ref[...] = (iota == idx[:, None]).astype(jnp.int32)

    @pl.when(i + NBUF < nc)
    def _next():
        start(i + NBUF)


def kernel(hidden_states, W):
    B, S, H = hidden_states.shape
    T = B * S
    x = hidden_states.reshape(T, H)
    grid = (T // CHUNK,)
    probs, onehot = pl.pallas_call(
        _router_body,
        grid=grid,
        in_specs=[
            pl.BlockSpec(memory_space=pltpu.MemorySpace.HBM),
            pl.BlockSpec((H, NUM_EXPERTS), lambda i: (0, 0)),
        ],
        out_specs=[
            pl.BlockSpec((CHUNK, 1), lambda i: (i, 0)),
            pl.BlockSpec((CHUNK, NUM_EXPERTS), lambda i: (i, 0)),
        ],
        out_shape=[
            jax.ShapeDtypeStruct((T, 1), jnp.float32),
            jax.ShapeDtypeStruct((T, NUM_EXPERTS), jnp.int32),
        ],
        scratch_shapes=[
            pltpu.VMEM((NBUF, CHUNK, HIDDEN), jnp.float32),
            pltpu.SemaphoreType.DMA((NBUF,)),
        ],
        compiler_params=pltpu.CompilerParams(
            dimension_semantics=("arbitrary",),
        ),
    )(x, W)
    p_out = probs.reshape(B, S, 1)
    oh_out = onehot.reshape(B, S, 1, NUM_EXPERTS).astype(jnp.int64)
    return (p_out, oh_out, p_out)
